# baseline (speedup 1.0000x reference)
import functools

import jax
import jax.numpy as jnp
from jax import lax
from jax.experimental import pallas as pl
from jax.experimental.pallas import tpu as pltpu
from jax.experimental.pallas import tpu_sc as plsc

NC = 2
NS = 16
NW = NC * NS
L = 16
N_IDX = 16384
EMB = 64
NVEC = EMB // L
CHUNK = 128
CHUNKS_PER_W = N_IDX // (NW * CHUNK)
ROWS_PER_STEP = 4


def _sc_partial_sums(effects2d, table):
    mesh = plsc.VectorSubcoreMesh(
        core_axis_name="c", subcore_axis_name="s",
        num_cores=NC, num_subcores=NS,
    )

    @functools.partial(
        pl.kernel,
        out_type=jax.ShapeDtypeStruct((NW, EMB), jnp.float32),
        mesh=mesh,
        scratch_types=[
            pltpu.VMEM((CHUNKS_PER_W, CHUNK), jnp.int32),
            pltpu.VMEM((2, CHUNK, EMB), jnp.float32),
            pltpu.VMEM((EMB,), jnp.float32),
            pltpu.SemaphoreType.DMA,
            pltpu.SemaphoreType.DMA,
        ],
        compiler_params=pltpu.CompilerParams(use_tc_tiling_on_sc=False),
    )
    def k(eff_hbm, tab_hbm, out_hbm, idx_v, rows_v, part_v, sem0, sem1):
        cid = lax.axis_index("c")
        sid = lax.axis_index("s")
        wid = sid * NC + cid

        pltpu.sync_copy(eff_hbm.at[pl.ds(wid * CHUNKS_PER_W, CHUNKS_PER_W)],
                        idx_v)

        sems = (sem0, sem1)
        copies = [None, None]
        copies[0] = pltpu.async_copy(tab_hbm.at[idx_v.at[0]], rows_v.at[0],
                                     sems[0])

        acc = [jnp.zeros((L,), jnp.float32) for _ in range(NVEC)]
        for j in range(CHUNKS_PER_W):
            buf = j % 2
            if j + 1 < CHUNKS_PER_W:
                nbuf = (j + 1) % 2
                copies[nbuf] = pltpu.async_copy(
                    tab_hbm.at[idx_v.at[j + 1]], rows_v.at[nbuf], sems[nbuf])
            copies[buf].wait()

            def body(i, a, _buf=buf):
                out = list(a)
                for u in range(ROWS_PER_STEP):
                    row = i * ROWS_PER_STEP + u
                    for q in range(NVEC):
                        out[q] = out[q] + rows_v[_buf, row, pl.ds(q * L, L)]
                return tuple(out)

            acc = list(lax.fori_loop(0, CHUNK // ROWS_PER_STEP, body,
                                     tuple(acc)))

        for q in range(NVEC):
            part_v[pl.ds(q * L, L)] = acc[q]
        pltpu.sync_copy(part_v, out_hbm.at[wid])

    return k(effects2d, table)


def _tc_finish(partials, W, b2d):
    def body(p_ref, w_ref, b_ref, o_ref):
        pooled = jnp.sum(p_ref[...], axis=0, keepdims=True) * (1.0 / N_IDX)
        o_ref[...] = lax.dot_general(
            pooled, w_ref[...], (((1,), (1,)), ((), ())),
            preferred_element_type=jnp.float32) + b_ref[...]

    return pl.pallas_call(
        body,
        out_shape=jax.ShapeDtypeStruct((1, EMB), jnp.float32),
    )(partials, W, b2d)


def kernel(effects, table, W, b):
    effects2d = effects.reshape(NW * CHUNKS_PER_W, CHUNK)
    partials = _sc_partial_sums(effects2d, table)
    out = _tc_finish(partials, W, b.reshape(1, EMB))
    return out.reshape(EMB)
